# Initial kernel scaffold; baseline (speedup 1.0000x reference)
import jax
import jax.numpy as jnp
from jax.experimental import pallas as pl


def kernel(x, edge_index, W1, b1, W2, b2):
    raise NotImplementedError("write your pallas kernel here")


# speedup vs baseline: 8.4102x; 8.4102x over previous
import functools

import jax
import jax.numpy as jnp
from jax import lax
from jax.experimental import pallas as pl
from jax.experimental.pallas import tpu as pltpu
from jax.experimental.pallas import tpu_sc as plsc

N = 10000
NP = 10240
E = 320000
K_IT = 10
ALPHA = 0.1
D1 = 64
D2 = 40
D2P = 48
RT = NP // 16
ET = E // 16
CH = 128
NCH = ET // CH
TAIL = ET - NCH * CH
ETD = E // 32
NCHD = ETD // CH
TAILD = ETD - NCHD * CH

_SC_PARAMS = pltpu.CompilerParams(use_tc_tiling_on_sc=False)


@functools.cache
def _mesh():
    return plsc.VectorSubcoreMesh(core_axis_name="c", subcore_axis_name="s")


def _zero_vmem(buf, n):
    z = jnp.zeros((16,), jnp.float32)

    def body(i, _):
        buf[pl.ds(i * 16, 16)] = z
        return 0

    lax.fori_loop(0, n // 16, body, 0)


def _deg_body(dst_hbm, out_hbm, idx_v, idx_t, ones_v, ones_t, acc, zbuf):
    c = lax.axis_index("c")
    s = lax.axis_index("s")

    _zero_vmem(zbuf, RT)
    o = jnp.ones((16,), jnp.float32)

    def fill(i, _):
        ones_v[pl.ds(i * 16, 16)] = o
        return 0

    lax.fori_loop(0, CH // 16, fill, 0)

    def fill_t(i, _):
        ones_t[pl.ds(i * 16, 16)] = o
        return 0

    lax.fori_loop(0, TAILD // 16, fill_t, 0)

    pltpu.sync_copy(zbuf, acc.at[pl.ds(s * RT, RT)])
    plsc.subcore_barrier()

    ebase = (s * 2 + c) * ETD

    def chunk(j, _):
        pltpu.sync_copy(dst_hbm.at[pl.ds(ebase + j * CH, CH)], idx_v)
        pltpu.sync_copy(ones_v, acc.at[idx_v], add=True)
        return 0

    lax.fori_loop(0, NCHD, chunk, 0)
    pltpu.sync_copy(dst_hbm.at[pl.ds(ebase + NCHD * CH, TAILD)], idx_t)
    pltpu.sync_copy(ones_t, acc.at[idx_t], add=True)
    plsc.subcore_barrier()

    pltpu.sync_copy(acc.at[pl.ds(s * RT, RT)], out_hbm.at[c, pl.ds(s * RT, RT)])


@functools.cache
def _deg_partials():
    return pl.kernel(
    _deg_body,
    out_type=jax.ShapeDtypeStruct((2, NP), jnp.float32),
    mesh=_mesh(),
    scratch_types=[
        pltpu.VMEM((CH,), jnp.int32),
        pltpu.VMEM((TAILD,), jnp.int32),
        pltpu.VMEM((CH,), jnp.float32),
        pltpu.VMEM((TAILD,), jnp.float32),
        pltpu.VMEM_SHARED((NP,), jnp.float32),
        pltpu.VMEM((RT,), jnp.float32),
    ],
    compiler_params=_SC_PARAMS,
    )


_BLK = 640


def _mlp_body(x_ref, w1_ref, b1_ref, w2_ref, b2_ref, degp_ref,
              z01_ref, z02_ref, g1_ref, g2_ref, d16_ref):
    xb = x_ref[...]
    h = lax.dot_general(xb, w1_ref[...], (((1,), (1,)), ((), ())),
                        preferred_element_type=jnp.float32)
    h = jnp.maximum(h + b1_ref[...], 0.0)
    h2 = lax.dot_general(h, w2_ref[...], (((1,), (1,)), ((), ())),
                         preferred_element_type=jnp.float32)
    h2 = h2 + b2_ref[...]
    h2p = jnp.concatenate(
        [h2, jnp.zeros((_BLK, D2P - D2), jnp.float32)], axis=1)
    deg = 1.0 + degp_ref[0, :] + degp_ref[1, :]
    dis = lax.rsqrt(deg).reshape(_BLK, 1)
    z01_ref[...] = ALPHA * h
    z02_ref[...] = ALPHA * h2p
    g1_ref[...] = dis * h
    g2_ref[...] = dis * h2p
    d16_ref[...] = jnp.broadcast_to(dis, (_BLK, 16))


def _mlp(x, W1, b1, W2, b2, degp):
    n_blk = NP // _BLK
    return pl.pallas_call(
        _mlp_body,
        grid=(n_blk,),
        in_specs=[
            pl.BlockSpec((_BLK, 128), lambda i: (i, 0)),
            pl.BlockSpec((D1, 128), lambda i: (0, 0)),
            pl.BlockSpec((1, D1), lambda i: (0, 0)),
            pl.BlockSpec((D2, D1), lambda i: (0, 0)),
            pl.BlockSpec((1, D2), lambda i: (0, 0)),
            pl.BlockSpec((2, _BLK), lambda i: (0, i)),
        ],
        out_specs=[
            pl.BlockSpec((_BLK, D1), lambda i: (i, 0)),
            pl.BlockSpec((_BLK, D2P), lambda i: (i, 0)),
            pl.BlockSpec((_BLK, D1), lambda i: (i, 0)),
            pl.BlockSpec((_BLK, D2P), lambda i: (i, 0)),
            pl.BlockSpec((_BLK, 16), lambda i: (i, 0)),
        ],
        out_shape=[
            jax.ShapeDtypeStruct((NP, D1), jnp.float32),
            jax.ShapeDtypeStruct((NP, D2P), jnp.float32),
            jax.ShapeDtypeStruct((NP, D1), jnp.float32),
            jax.ShapeDtypeStruct((NP, D2P), jnp.float32),
            jax.ShapeDtypeStruct((NP, 16), jnp.float32),
        ],
    )(x, W1, b1, W2, b2, degp)


def _seed_phase(g_hbm, acc, rbase):
    pltpu.sync_copy(g_hbm.at[pl.ds(rbase, RT)], acc.at[pl.ds(rbase, RT)])


def _scatter_phase(src_hbm, dst_hbm, g_hbm, acc, sbuf, dbuf, rows,
                   sbuf_t, dbuf_t, rows_t, ebase):
    def chunk(j, _):
        off = ebase + j * CH
        pltpu.sync_copy(src_hbm.at[pl.ds(off, CH)], sbuf)
        pltpu.sync_copy(dst_hbm.at[pl.ds(off, CH)], dbuf)
        pltpu.sync_copy(g_hbm.at[sbuf], rows)
        pltpu.sync_copy(rows, acc.at[dbuf], add=True)
        return 0

    lax.fori_loop(0, NCH, chunk, 0)
    off = ebase + NCH * CH
    pltpu.sync_copy(src_hbm.at[pl.ds(off, TAIL)], sbuf_t)
    pltpu.sync_copy(dst_hbm.at[pl.ds(off, TAIL)], dbuf_t)
    pltpu.sync_copy(g_hbm.at[sbuf_t], rows_t)
    pltpu.sync_copy(rows_t, acc.at[dbuf_t], add=True)


def _combine_phase(acc, z0_hbm, d16_hbm, g_hbm, abuf, z0c, d16c,
                   rbase, is_last, d):
    n_rc = RT // CH

    def rchunk(j, _):
        pltpu.sync_copy(acc.at[pl.ds(rbase + j * CH, CH)], abuf)
        pltpu.sync_copy(z0_hbm.at[pl.ds(rbase + j * CH, CH)], z0c)
        pltpu.sync_copy(d16_hbm.at[pl.ds(rbase + j * CH, CH)], d16c)

        def row(r, _):
            vdis = d16c[r, :]

            def col(cc, _):
                va = abuf[r, pl.ds(cc * 16, 16)]
                vz = z0c[r, pl.ds(cc * 16, 16)]
                vh = (1.0 - ALPHA) * (vdis * va) + vz
                vg = vdis * vh
                abuf[r, pl.ds(cc * 16, 16)] = jnp.where(is_last, vh, vg)
                return 0

            lax.fori_loop(0, d // 16, col, 0)
            return 0

        lax.fori_loop(0, CH, row, 0)
        pltpu.sync_copy(abuf, g_hbm.at[pl.ds(rbase + j * CH, CH)])
        return 0

    lax.fori_loop(0, n_rc, rchunk, 0)


def _appnp_body(src_hbm, dst_hbm, z01_hbm, z02_hbm, g1_hbm, g2_hbm, d16_hbm,
                out1_hbm, out2_hbm,
                sbuf, dbuf, sbuf_t, dbuf_t,
                rows1, rows1_t, rows2, rows2_t,
                z0c1, z0c2, d16c,
                acc1, acc2):
    c = lax.axis_index("c")
    s = lax.axis_index("s")
    rbase = s * RT
    ebase = s * ET
    on0 = c == 0
    on1 = c == 1

    @pl.when(on0)
    def _():
        def rchunk(j, _):
            pltpu.sync_copy(g1_hbm.at[pl.ds(rbase + j * CH, CH)], rows1)
            pltpu.sync_copy(rows1, out1_hbm.at[pl.ds(rbase + j * CH, CH)])
            return 0

        lax.fori_loop(0, RT // CH, rchunk, 0)

    @pl.when(on1)
    def _():
        def rchunk(j, _):
            pltpu.sync_copy(g2_hbm.at[pl.ds(rbase + j * CH, CH)], rows2)
            pltpu.sync_copy(rows2, out2_hbm.at[pl.ds(rbase + j * CH, CH)])
            return 0

        lax.fori_loop(0, RT // CH, rchunk, 0)

    def iteration(it, _):
        is_last = it == (K_IT - 1)

        @pl.when(on0)
        def _():
            _seed_phase(out1_hbm, acc1, rbase)

        @pl.when(on1)
        def _():
            _seed_phase(out2_hbm, acc2, rbase)

        plsc.subcore_barrier()

        @pl.when(on0)
        def _():
            _scatter_phase(src_hbm, dst_hbm, out1_hbm, acc1, sbuf, dbuf,
                           rows1, sbuf_t, dbuf_t, rows1_t, ebase)

        @pl.when(on1)
        def _():
            _scatter_phase(src_hbm, dst_hbm, out2_hbm, acc2, sbuf, dbuf,
                           rows2, sbuf_t, dbuf_t, rows2_t, ebase)

        plsc.subcore_barrier()

        @pl.when(on0)
        def _():
            _combine_phase(acc1, z01_hbm, d16_hbm, out1_hbm, rows1,
                           z0c1, d16c, rbase, is_last, D1)

        @pl.when(on1)
        def _():
            _combine_phase(acc2, z02_hbm, d16_hbm, out2_hbm, rows2,
                           z0c2, d16c, rbase, is_last, D2P)

        plsc.subcore_barrier()
        return 0

    lax.fori_loop(0, K_IT, iteration, 0)


@functools.cache
def _appnp():
    return pl.kernel(
    _appnp_body,
    out_type=(
        jax.ShapeDtypeStruct((NP, D1), jnp.float32),
        jax.ShapeDtypeStruct((NP, D2P), jnp.float32),
    ),
    mesh=_mesh(),
    scratch_types=[
        pltpu.VMEM((CH,), jnp.int32),
        pltpu.VMEM((CH,), jnp.int32),
        pltpu.VMEM((TAIL,), jnp.int32),
        pltpu.VMEM((TAIL,), jnp.int32),
        pltpu.VMEM((CH, D1), jnp.float32),
        pltpu.VMEM((TAIL, D1), jnp.float32),
        pltpu.VMEM((CH, D2P), jnp.float32),
        pltpu.VMEM((TAIL, D2P), jnp.float32),
        pltpu.VMEM((CH, D1), jnp.float32),
        pltpu.VMEM((CH, D2P), jnp.float32),
        pltpu.VMEM((CH, 16), jnp.float32),
        pltpu.VMEM_SHARED((NP, D1), jnp.float32),
        pltpu.VMEM_SHARED((NP, D2P), jnp.float32),
    ],
    compiler_params=_SC_PARAMS,
    )


def kernel(x, edge_index, W1, b1, W2, b2):
    src = edge_index[0].astype(jnp.int32)
    dst = edge_index[1].astype(jnp.int32)
    xp = jnp.pad(x, ((0, NP - N), (0, 0)))
    degp = _deg_partials()(dst)
    z01, z02, g1, g2, d16 = _mlp(xp, W1, b1.reshape(1, D1), W2,
                                 b2.reshape(1, D2), degp)
    out1p, out2p = _appnp()(src, dst, z01, z02, g1, g2, d16)
    return (x, out1p[:N], out2p[:N, :D2])


# speedup vs baseline: 13.2114x; 1.5709x over previous
import functools

import jax
import jax.numpy as jnp
from jax import lax
from jax.experimental import pallas as pl
from jax.experimental.pallas import tpu as pltpu
from jax.experimental.pallas import tpu_sc as plsc

N = 10000
NP = 10240
E = 320000
K_IT = 10
ALPHA = 0.1
D1 = 64
D2 = 40
D2P = 48
RT = NP // 16
ET = E // 16
CH = 128
NCH = ET // CH
TAIL = ET - NCH * CH
ETD = E // 32
NCHD = ETD // CH
TAILD = ETD - NCHD * CH

_SC_PARAMS = pltpu.CompilerParams(use_tc_tiling_on_sc=False)


@functools.cache
def _mesh():
    return plsc.VectorSubcoreMesh(core_axis_name="c", subcore_axis_name="s")


def _zero_vmem(buf, n):
    z = jnp.zeros((16,), jnp.float32)

    def body(i, _):
        buf[pl.ds(i * 16, 16)] = z
        return 0

    lax.fori_loop(0, n // 16, body, 0)


def _deg_body(dst_hbm, out_hbm, idx_v, idx_t, ones_v, ones_t, acc, zbuf):
    c = lax.axis_index("c")
    s = lax.axis_index("s")

    _zero_vmem(zbuf, RT)
    o = jnp.ones((16,), jnp.float32)

    def fill(i, _):
        ones_v[pl.ds(i * 16, 16)] = o
        return 0

    lax.fori_loop(0, CH // 16, fill, 0)

    def fill_t(i, _):
        ones_t[pl.ds(i * 16, 16)] = o
        return 0

    lax.fori_loop(0, TAILD // 16, fill_t, 0)

    pltpu.sync_copy(zbuf, acc.at[pl.ds(s * RT, RT)])
    plsc.subcore_barrier()

    ebase = (s * 2 + c) * ETD

    def chunk(j, _):
        pltpu.sync_copy(dst_hbm.at[pl.ds(ebase + j * CH, CH)], idx_v)
        pltpu.sync_copy(ones_v, acc.at[idx_v], add=True)
        return 0

    lax.fori_loop(0, NCHD, chunk, 0)
    pltpu.sync_copy(dst_hbm.at[pl.ds(ebase + NCHD * CH, TAILD)], idx_t)
    pltpu.sync_copy(ones_t, acc.at[idx_t], add=True)
    plsc.subcore_barrier()

    pltpu.sync_copy(acc.at[pl.ds(s * RT, RT)], out_hbm.at[c, pl.ds(s * RT, RT)])


@functools.cache
def _deg_partials():
    return pl.kernel(
    _deg_body,
    out_type=jax.ShapeDtypeStruct((2, NP), jnp.float32),
    mesh=_mesh(),
    scratch_types=[
        pltpu.VMEM((CH,), jnp.int32),
        pltpu.VMEM((TAILD,), jnp.int32),
        pltpu.VMEM((CH,), jnp.float32),
        pltpu.VMEM((TAILD,), jnp.float32),
        pltpu.VMEM_SHARED((NP,), jnp.float32),
        pltpu.VMEM((RT,), jnp.float32),
    ],
    compiler_params=_SC_PARAMS,
    )


_BLK = 640


def _mlp_body(x_ref, w1_ref, b1_ref, w2_ref, b2_ref, degp_ref,
              z01_ref, z02_ref, g1_ref, g2_ref, d16_ref):
    xb = x_ref[...]
    h = lax.dot_general(xb, w1_ref[...], (((1,), (1,)), ((), ())),
                        preferred_element_type=jnp.float32)
    h = jnp.maximum(h + b1_ref[...], 0.0)
    h2 = lax.dot_general(h, w2_ref[...], (((1,), (1,)), ((), ())),
                         preferred_element_type=jnp.float32)
    h2 = h2 + b2_ref[...]
    h2p = jnp.concatenate(
        [h2, jnp.zeros((_BLK, D2P - D2), jnp.float32)], axis=1)
    deg = 1.0 + degp_ref[0, :] + degp_ref[1, :]
    dis = lax.rsqrt(deg).reshape(_BLK, 1)
    z01_ref[...] = ALPHA * h
    z02_ref[...] = ALPHA * h2p
    g1_ref[...] = dis * h
    g2_ref[...] = dis * h2p
    d16_ref[...] = jnp.broadcast_to(dis, (_BLK, 16))


def _mlp(x, W1, b1, W2, b2, degp):
    n_blk = NP // _BLK
    return pl.pallas_call(
        _mlp_body,
        grid=(n_blk,),
        in_specs=[
            pl.BlockSpec((_BLK, 128), lambda i: (i, 0)),
            pl.BlockSpec((D1, 128), lambda i: (0, 0)),
            pl.BlockSpec((1, D1), lambda i: (0, 0)),
            pl.BlockSpec((D2, D1), lambda i: (0, 0)),
            pl.BlockSpec((1, D2), lambda i: (0, 0)),
            pl.BlockSpec((2, _BLK), lambda i: (0, i)),
        ],
        out_specs=[
            pl.BlockSpec((_BLK, D1), lambda i: (i, 0)),
            pl.BlockSpec((_BLK, D2P), lambda i: (i, 0)),
            pl.BlockSpec((_BLK, D1), lambda i: (i, 0)),
            pl.BlockSpec((_BLK, D2P), lambda i: (i, 0)),
            pl.BlockSpec((_BLK, 16), lambda i: (i, 0)),
        ],
        out_shape=[
            jax.ShapeDtypeStruct((NP, D1), jnp.float32),
            jax.ShapeDtypeStruct((NP, D2P), jnp.float32),
            jax.ShapeDtypeStruct((NP, D1), jnp.float32),
            jax.ShapeDtypeStruct((NP, D2P), jnp.float32),
            jax.ShapeDtypeStruct((NP, 16), jnp.float32),
        ],
    )(x, W1, b1, W2, b2, degp)


def _seed_phase(g_hbm, acc, rbase):
    pltpu.sync_copy(g_hbm.at[pl.ds(rbase, RT)], acc.at[pl.ds(rbase, RT)])


def _scatter_phase(src_hbm, dst_hbm, g_hbm, acc, s0, d0, s1, d1, r0, r1,
                   semA, semB, sbuf_t, dbuf_t, rows_t, ebase):
    npair = NCH // 2

    pltpu.sync_copy(src_hbm.at[pl.ds(ebase, CH)], s0)
    pltpu.sync_copy(dst_hbm.at[pl.ds(ebase, CH)], d0)
    pltpu.async_copy(g_hbm.at[s0], r0, semA)

    def pair(t, _):
        b_off = ebase + (2 * t) * CH + CH
        n_off = b_off + CH
        pltpu.sync_copy(src_hbm.at[pl.ds(b_off, CH)], s1)
        pltpu.sync_copy(dst_hbm.at[pl.ds(b_off, CH)], d1)
        pltpu.make_async_copy(g_hbm.at[s0], r0, semA).wait()
        pltpu.async_copy(g_hbm.at[s1], r1, semB)
        pltpu.sync_copy(r0, acc.at[d0], add=True)

        @pl.when(t < npair - 1)
        def _():
            pltpu.sync_copy(src_hbm.at[pl.ds(n_off, CH)], s0)
            pltpu.sync_copy(dst_hbm.at[pl.ds(n_off, CH)], d0)

        pltpu.make_async_copy(g_hbm.at[s1], r1, semB).wait()

        @pl.when(t < npair - 1)
        def _():
            pltpu.async_copy(g_hbm.at[s0], r0, semA)

        pltpu.sync_copy(r1, acc.at[d1], add=True)
        return 0

    lax.fori_loop(0, npair, pair, 0)
    off = ebase + NCH * CH
    pltpu.sync_copy(src_hbm.at[pl.ds(off, TAIL)], sbuf_t)
    pltpu.sync_copy(dst_hbm.at[pl.ds(off, TAIL)], dbuf_t)
    pltpu.sync_copy(g_hbm.at[sbuf_t], rows_t)
    pltpu.sync_copy(rows_t, acc.at[dbuf_t], add=True)


def _combine_phase(acc, z0_hbm, d16_hbm, g_hbm, abuf, z0c, d16c,
                   rbase, is_last, d):
    n_rc = RT // CH

    def rchunk(j, _):
        pltpu.sync_copy(acc.at[pl.ds(rbase + j * CH, CH)], abuf)
        pltpu.sync_copy(z0_hbm.at[pl.ds(rbase + j * CH, CH)], z0c)
        pltpu.sync_copy(d16_hbm.at[pl.ds(rbase + j * CH, CH)], d16c)

        def row(r, _):
            vdis = d16c[r, :]

            def col(cc, _):
                va = abuf[r, pl.ds(cc * 16, 16)]
                vz = z0c[r, pl.ds(cc * 16, 16)]
                vh = (1.0 - ALPHA) * (vdis * va) + vz
                vg = vdis * vh
                abuf[r, pl.ds(cc * 16, 16)] = jnp.where(is_last, vh, vg)
                return 0

            lax.fori_loop(0, d // 16, col, 0)
            return 0

        lax.fori_loop(0, CH, row, 0)
        pltpu.sync_copy(abuf, g_hbm.at[pl.ds(rbase + j * CH, CH)])
        return 0

    lax.fori_loop(0, n_rc, rchunk, 0)


def _appnp_body(src_hbm, dst_hbm, z01_hbm, z02_hbm, g1_hbm, g2_hbm, d16_hbm,
                out1_hbm, out2_hbm,
                sbuf, dbuf, sbuf1, dbuf1, sbuf_t, dbuf_t,
                rows1, rows1b, rows1_t, rows2, rows2b, rows2_t,
                z0c1, z0c2, d16c, semA, semB,
                acc1, acc2):
    c = lax.axis_index("c")
    s = lax.axis_index("s")
    rbase = s * RT
    ebase = s * ET
    on0 = c == 0
    on1 = c == 1

    @pl.when(on0)
    def _():
        def rchunk(j, _):
            pltpu.sync_copy(g1_hbm.at[pl.ds(rbase + j * CH, CH)], rows1)
            pltpu.sync_copy(rows1, out1_hbm.at[pl.ds(rbase + j * CH, CH)])
            return 0

        lax.fori_loop(0, RT // CH, rchunk, 0)

    @pl.when(on1)
    def _():
        def rchunk(j, _):
            pltpu.sync_copy(g2_hbm.at[pl.ds(rbase + j * CH, CH)], rows2)
            pltpu.sync_copy(rows2, out2_hbm.at[pl.ds(rbase + j * CH, CH)])
            return 0

        lax.fori_loop(0, RT // CH, rchunk, 0)

    def iteration(it, _):
        is_last = it == (K_IT - 1)

        @pl.when(on0)
        def _():
            _seed_phase(out1_hbm, acc1, rbase)

        @pl.when(on1)
        def _():
            _seed_phase(out2_hbm, acc2, rbase)

        plsc.subcore_barrier()

        @pl.when(on0)
        def _():
            _scatter_phase(src_hbm, dst_hbm, out1_hbm, acc1, sbuf, dbuf,
                           sbuf1, dbuf1, rows1, rows1b, semA, semB,
                           sbuf_t, dbuf_t, rows1_t, ebase)

        @pl.when(on1)
        def _():
            _scatter_phase(src_hbm, dst_hbm, out2_hbm, acc2, sbuf, dbuf,
                           sbuf1, dbuf1, rows2, rows2b, semA, semB,
                           sbuf_t, dbuf_t, rows2_t, ebase)

        plsc.subcore_barrier()

        @pl.when(on0)
        def _():
            _combine_phase(acc1, z01_hbm, d16_hbm, out1_hbm, rows1,
                           z0c1, d16c, rbase, is_last, D1)

        @pl.when(on1)
        def _():
            _combine_phase(acc2, z02_hbm, d16_hbm, out2_hbm, rows2,
                           z0c2, d16c, rbase, is_last, D2P)

        plsc.subcore_barrier()
        return 0

    lax.fori_loop(0, K_IT, iteration, 0)


@functools.cache
def _appnp():
    return pl.kernel(
    _appnp_body,
    out_type=(
        jax.ShapeDtypeStruct((NP, D1), jnp.float32),
        jax.ShapeDtypeStruct((NP, D2P), jnp.float32),
    ),
    mesh=_mesh(),
    scratch_types=[
        pltpu.VMEM((CH,), jnp.int32),
        pltpu.VMEM((CH,), jnp.int32),
        pltpu.VMEM((CH,), jnp.int32),
        pltpu.VMEM((CH,), jnp.int32),
        pltpu.VMEM((TAIL,), jnp.int32),
        pltpu.VMEM((TAIL,), jnp.int32),
        pltpu.VMEM((CH, D1), jnp.float32),
        pltpu.VMEM((CH, D1), jnp.float32),
        pltpu.VMEM((TAIL, D1), jnp.float32),
        pltpu.VMEM((CH, D2P), jnp.float32),
        pltpu.VMEM((CH, D2P), jnp.float32),
        pltpu.VMEM((TAIL, D2P), jnp.float32),
        pltpu.VMEM((CH, D1), jnp.float32),
        pltpu.VMEM((CH, D2P), jnp.float32),
        pltpu.VMEM((CH, 16), jnp.float32),
        pltpu.SemaphoreType.DMA,
        pltpu.SemaphoreType.DMA,
        pltpu.VMEM_SHARED((NP, D1), jnp.float32),
        pltpu.VMEM_SHARED((NP, D2P), jnp.float32),
    ],
    compiler_params=_SC_PARAMS,
    )


def kernel(x, edge_index, W1, b1, W2, b2):
    src = edge_index[0].astype(jnp.int32)
    dst = edge_index[1].astype(jnp.int32)
    xp = jnp.pad(x, ((0, NP - N), (0, 0)))
    degp = _deg_partials()(dst)
    z01, z02, g1, g2, d16 = _mlp(xp, W1, b1.reshape(1, D1), W2,
                                 b2.reshape(1, D2), degp)
    out1p, out2p = _appnp()(src, dst, z01, z02, g1, g2, d16)
    return (x, out1p[:N], out2p[:N, :D2])


# speedup vs baseline: 19.8857x; 1.5052x over previous
import functools

import jax
import jax.numpy as jnp
from jax import lax
from jax.experimental import pallas as pl
from jax.experimental.pallas import tpu as pltpu
from jax.experimental.pallas import tpu_sc as plsc

N = 10000
NP = 10240
E = 320000
K_IT = 10
ALPHA = 0.1
D1 = 64
D2 = 40
D2P = 48
RT = NP // 16
ET = E // 16
CH = 128
NCH = ET // CH
TAIL = ET - NCH * CH
ETD = E // 32
NCHD = ETD // CH
TAILD = ETD - NCHD * CH

_SC_PARAMS = pltpu.CompilerParams(use_tc_tiling_on_sc=False)


@functools.cache
def _mesh():
    return plsc.VectorSubcoreMesh(core_axis_name="c", subcore_axis_name="s")


def _zero_vmem(buf, n):
    z = jnp.zeros((16,), jnp.float32)

    def body(i, _):
        buf[pl.ds(i * 16, 16)] = z
        return 0

    lax.fori_loop(0, n // 16, body, 0)


def _deg_body(dst_hbm, out_hbm, idx_v, idx_t, ones_v, ones_t, acc, zbuf):
    c = lax.axis_index("c")
    s = lax.axis_index("s")

    _zero_vmem(zbuf, RT)
    o = jnp.ones((16,), jnp.float32)

    def fill(i, _):
        ones_v[pl.ds(i * 16, 16)] = o
        return 0

    lax.fori_loop(0, CH // 16, fill, 0)

    def fill_t(i, _):
        ones_t[pl.ds(i * 16, 16)] = o
        return 0

    lax.fori_loop(0, TAILD // 16, fill_t, 0)

    pltpu.sync_copy(zbuf, acc.at[pl.ds(s * RT, RT)])
    plsc.subcore_barrier()

    ebase = (s * 2 + c) * ETD

    def chunk(j, _):
        pltpu.sync_copy(dst_hbm.at[pl.ds(ebase + j * CH, CH)], idx_v)
        pltpu.sync_copy(ones_v, acc.at[idx_v], add=True)
        return 0

    lax.fori_loop(0, NCHD, chunk, 0)
    pltpu.sync_copy(dst_hbm.at[pl.ds(ebase + NCHD * CH, TAILD)], idx_t)
    pltpu.sync_copy(ones_t, acc.at[idx_t], add=True)
    plsc.subcore_barrier()

    pltpu.sync_copy(acc.at[pl.ds(s * RT, RT)], out_hbm.at[c, pl.ds(s * RT, RT)])


@functools.cache
def _deg_partials():
    return pl.kernel(
    _deg_body,
    out_type=jax.ShapeDtypeStruct((2, NP), jnp.float32),
    mesh=_mesh(),
    scratch_types=[
        pltpu.VMEM((CH,), jnp.int32),
        pltpu.VMEM((TAILD,), jnp.int32),
        pltpu.VMEM((CH,), jnp.float32),
        pltpu.VMEM((TAILD,), jnp.float32),
        pltpu.VMEM_SHARED((NP,), jnp.float32),
        pltpu.VMEM((RT,), jnp.float32),
    ],
    compiler_params=_SC_PARAMS,
    )


_BLK = 640


def _mlp_body(x_ref, w1_ref, b1_ref, w2_ref, b2_ref, degp_ref,
              z01_ref, z02_ref, g1_ref, g2_ref, d16_ref):
    xb = x_ref[...]
    h = lax.dot_general(xb, w1_ref[...], (((1,), (1,)), ((), ())),
                        preferred_element_type=jnp.float32)
    h = jnp.maximum(h + b1_ref[...], 0.0)
    h2 = lax.dot_general(h, w2_ref[...], (((1,), (1,)), ((), ())),
                         preferred_element_type=jnp.float32)
    h2 = h2 + b2_ref[...]
    h2p = jnp.concatenate(
        [h2, jnp.zeros((_BLK, D2P - D2), jnp.float32)], axis=1)
    deg = 1.0 + degp_ref[0, :] + degp_ref[1, :]
    dis = lax.rsqrt(deg).reshape(_BLK, 1)
    z01_ref[...] = ALPHA * h
    z02_ref[...] = ALPHA * h2p
    g1_ref[...] = dis * h
    g2_ref[...] = dis * h2p
    d16_ref[...] = jnp.broadcast_to(dis, (_BLK, 16))


def _mlp(x, W1, b1, W2, b2, degp):
    n_blk = NP // _BLK
    return pl.pallas_call(
        _mlp_body,
        grid=(n_blk,),
        in_specs=[
            pl.BlockSpec((_BLK, 128), lambda i: (i, 0)),
            pl.BlockSpec((D1, 128), lambda i: (0, 0)),
            pl.BlockSpec((1, D1), lambda i: (0, 0)),
            pl.BlockSpec((D2, D1), lambda i: (0, 0)),
            pl.BlockSpec((1, D2), lambda i: (0, 0)),
            pl.BlockSpec((2, _BLK), lambda i: (0, i)),
        ],
        out_specs=[
            pl.BlockSpec((_BLK, D1), lambda i: (i, 0)),
            pl.BlockSpec((_BLK, D2P), lambda i: (i, 0)),
            pl.BlockSpec((_BLK, D1), lambda i: (i, 0)),
            pl.BlockSpec((_BLK, D2P), lambda i: (i, 0)),
            pl.BlockSpec((_BLK, 16), lambda i: (i, 0)),
        ],
        out_shape=[
            jax.ShapeDtypeStruct((NP, D1), jnp.float32),
            jax.ShapeDtypeStruct((NP, D2P), jnp.float32),
            jax.ShapeDtypeStruct((NP, D1), jnp.float32),
            jax.ShapeDtypeStruct((NP, D2P), jnp.float32),
            jax.ShapeDtypeStruct((NP, 16), jnp.float32),
        ],
    )(x, W1, b1, W2, b2, degp)


def _seed_phase(g_hbm, acc, rbase):
    pltpu.sync_copy(g_hbm.at[pl.ds(rbase, RT)], acc.at[pl.ds(rbase, RT)])


def _copy_idx(src_v, dst_v):
    def cp(i, _):
        dst_v[pl.ds(i * 16, 16)] = src_v[pl.ds(i * 16, 16)]
        return 0

    lax.fori_loop(0, CH // 16, cp, 0)


def _scatter_phase(src_hbm, dst_hbm, g_hbm, acc,
                   ss, dd, sp, dp, rr, semg, sems, semi,
                   sbuf_t, dbuf_t, rows_t, ebase):
    ntri = NCH // 3

    for k in range(3):
        off = ebase + k * CH
        pltpu.async_copy(src_hbm.at[pl.ds(off, CH)], sp[k], semi[k])
        pltpu.async_copy(dst_hbm.at[pl.ds(off, CH)], dp[k], semi[k])
    for k in range(3):
        off = ebase + k * CH
        pltpu.make_async_copy(src_hbm.at[pl.ds(off, CH)], sp[k],
                              semi[k]).wait()
        pltpu.make_async_copy(dst_hbm.at[pl.ds(off, CH)], dp[k],
                              semi[k]).wait()
        _copy_idx(sp[k], ss[k])
        pltpu.async_copy(g_hbm.at[ss[k]], rr[k], semg[k])

    def tri(u, _):
        c0 = ebase + (3 * u) * CH

        for k in range(3):
            pltpu.make_async_copy(g_hbm.at[ss[k]], rr[k], semg[k]).wait()
            _copy_idx(dp[k], dd[k])
            pltpu.async_copy(rr[k], acc.at[dd[k]], sems[k], add=True)

        @pl.when(u < ntri - 1)
        def _():
            for k in range(3):
                off = c0 + (3 + k) * CH
                pltpu.async_copy(src_hbm.at[pl.ds(off, CH)], sp[k], semi[k])
                pltpu.async_copy(dst_hbm.at[pl.ds(off, CH)], dp[k], semi[k])

        for k in range(3):
            pltpu.make_async_copy(rr[k], acc.at[dd[k]], sems[k]).wait()

            @pl.when(u < ntri - 1)
            def _():
                off = c0 + (3 + k) * CH
                pltpu.make_async_copy(src_hbm.at[pl.ds(off, CH)], sp[k],
                                      semi[k]).wait()
                pltpu.make_async_copy(dst_hbm.at[pl.ds(off, CH)], dp[k],
                                      semi[k]).wait()
                _copy_idx(sp[k], ss[k])
                pltpu.async_copy(g_hbm.at[ss[k]], rr[k], semg[k])

        return 0

    lax.fori_loop(0, ntri, tri, 0)
    off = ebase + NCH * CH
    pltpu.sync_copy(src_hbm.at[pl.ds(off, TAIL)], sbuf_t)
    pltpu.sync_copy(dst_hbm.at[pl.ds(off, TAIL)], dbuf_t)
    pltpu.sync_copy(g_hbm.at[sbuf_t], rows_t)
    pltpu.sync_copy(rows_t, acc.at[dbuf_t], add=True)


def _combine_phase(acc, z0_hbm, d16_hbm, g_hbm, abuf, z0c, d16c,
                   rbase, is_last, d):
    n_rc = RT // CH

    def rchunk(j, _):
        pltpu.sync_copy(acc.at[pl.ds(rbase + j * CH, CH)], abuf)
        pltpu.sync_copy(z0_hbm.at[pl.ds(rbase + j * CH, CH)], z0c)
        pltpu.sync_copy(d16_hbm.at[pl.ds(rbase + j * CH, CH)], d16c)

        def row(r, _):
            vdis = d16c[r, :]

            def col(cc, _):
                va = abuf[r, pl.ds(cc * 16, 16)]
                vz = z0c[r, pl.ds(cc * 16, 16)]
                vh = (1.0 - ALPHA) * (vdis * va) + vz
                vg = vdis * vh
                abuf[r, pl.ds(cc * 16, 16)] = jnp.where(is_last, vh, vg)
                return 0

            lax.fori_loop(0, d // 16, col, 0)
            return 0

        lax.fori_loop(0, CH, row, 0)
        pltpu.sync_copy(abuf, g_hbm.at[pl.ds(rbase + j * CH, CH)])
        return 0

    lax.fori_loop(0, n_rc, rchunk, 0)


def _appnp_body(src_hbm, dst_hbm, z01_hbm, z02_hbm, g1_hbm, g2_hbm, d16_hbm,
                out1_hbm, out2_hbm,
                ss0, ss1, ss2, dd0, dd1, dd2,
                sp0, sp1, sp2, dp0, dp1, dp2,
                rA0, rA1, rA2, rB0, rB1, rB2,
                sbuf_t, dbuf_t, rows1_t, rows2_t, d16c,
                sg0, sg1, sg2, sc0, sc1, sc2, si0, si1, si2,
                acc1, acc2):
    ss = [ss0, ss1, ss2]
    dd = [dd0, dd1, dd2]
    sp = [sp0, sp1, sp2]
    dp = [dp0, dp1, dp2]
    rA = [rA0, rA1, rA2]
    rB = [rB0, rB1, rB2]
    semg = [sg0, sg1, sg2]
    sems = [sc0, sc1, sc2]
    semi = [si0, si1, si2]
    c = lax.axis_index("c")
    s = lax.axis_index("s")
    rbase = s * RT
    ebase = s * ET
    on0 = c == 0
    on1 = c == 1

    @pl.when(on0)
    def _():
        def rchunk(j, _):
            pltpu.sync_copy(g1_hbm.at[pl.ds(rbase + j * CH, CH)], rA0)
            pltpu.sync_copy(rA0, out1_hbm.at[pl.ds(rbase + j * CH, CH)])
            return 0

        lax.fori_loop(0, RT // CH, rchunk, 0)

    @pl.when(on1)
    def _():
        def rchunk(j, _):
            pltpu.sync_copy(g2_hbm.at[pl.ds(rbase + j * CH, CH)], rB0)
            pltpu.sync_copy(rB0, out2_hbm.at[pl.ds(rbase + j * CH, CH)])
            return 0

        lax.fori_loop(0, RT // CH, rchunk, 0)

    def iteration(it, _):
        is_last = it == (K_IT - 1)

        @pl.when(on0)
        def _():
            _seed_phase(out1_hbm, acc1, rbase)

        @pl.when(on1)
        def _():
            _seed_phase(out2_hbm, acc2, rbase)

        plsc.subcore_barrier()

        @pl.when(on0)
        def _():
            _scatter_phase(src_hbm, dst_hbm, out1_hbm, acc1,
                           ss, dd, sp, dp, rA, semg, sems, semi,
                           sbuf_t, dbuf_t, rows1_t, ebase)

        @pl.when(on1)
        def _():
            _scatter_phase(src_hbm, dst_hbm, out2_hbm, acc2,
                           ss, dd, sp, dp, rB, semg, sems, semi,
                           sbuf_t, dbuf_t, rows2_t, ebase)

        plsc.subcore_barrier()

        @pl.when(on0)
        def _():
            _combine_phase(acc1, z01_hbm, d16_hbm, out1_hbm, rA0,
                           rA1, d16c, rbase, is_last, D1)

        @pl.when(on1)
        def _():
            _combine_phase(acc2, z02_hbm, d16_hbm, out2_hbm, rB0,
                           rB1, d16c, rbase, is_last, D2P)

        plsc.subcore_barrier()
        return 0

    lax.fori_loop(0, K_IT, iteration, 0)


@functools.cache
def _appnp():
    return pl.kernel(
    _appnp_body,
    out_type=(
        jax.ShapeDtypeStruct((NP, D1), jnp.float32),
        jax.ShapeDtypeStruct((NP, D2P), jnp.float32),
    ),
    mesh=_mesh(),
    scratch_types=(
        [pltpu.VMEM((CH,), jnp.int32)] * 12
        + [pltpu.VMEM((CH, D1), jnp.float32)] * 3
        + [pltpu.VMEM((CH, D2P), jnp.float32)] * 3
        + [
            pltpu.VMEM((TAIL,), jnp.int32),
            pltpu.VMEM((TAIL,), jnp.int32),
            pltpu.VMEM((TAIL, D1), jnp.float32),
            pltpu.VMEM((TAIL, D2P), jnp.float32),
            pltpu.VMEM((CH, 16), jnp.float32),
        ]
        + [pltpu.SemaphoreType.DMA] * 9
        + [
            pltpu.VMEM_SHARED((NP, D1), jnp.float32),
            pltpu.VMEM_SHARED((NP, D2P), jnp.float32),
        ]
    ),
    compiler_params=_SC_PARAMS,
    )


def kernel(x, edge_index, W1, b1, W2, b2):
    src = edge_index[0].astype(jnp.int32)
    dst = edge_index[1].astype(jnp.int32)
    xp = jnp.pad(x, ((0, NP - N), (0, 0)))
    degp = _deg_partials()(dst)
    z01, z02, g1, g2, d16 = _mlp(xp, W1, b1.reshape(1, D1), W2,
                                 b2.reshape(1, D2), degp)
    out1p, out2p = _appnp()(src, dst, z01, z02, g1, g2, d16)
    return (x, out1p[:N], out2p[:N, :D2])


# speedup vs baseline: 23.5691x; 1.1852x over previous
import functools

import jax
import jax.numpy as jnp
from jax import lax
from jax.experimental import pallas as pl
from jax.experimental.pallas import tpu as pltpu
from jax.experimental.pallas import tpu_sc as plsc

N = 10000
NP = 10240
E = 320000
K_IT = 10
ALPHA = 0.1
D1 = 64
D2 = 40
D2P = 48
RT = NP // 16
ET = E // 16
CH = 128
NCH = ET // CH
TAIL = ET - NCH * CH
ETD = E // 32
NCHD = ETD // CH
TAILD = ETD - NCHD * CH

_SC_PARAMS = pltpu.CompilerParams(use_tc_tiling_on_sc=False)


@functools.cache
def _mesh():
    return plsc.VectorSubcoreMesh(core_axis_name="c", subcore_axis_name="s")


def _zero_vmem(buf, n):
    z = jnp.zeros((16,), jnp.float32)

    def body(i, _):
        buf[pl.ds(i * 16, 16)] = z
        return 0

    lax.fori_loop(0, n // 16, body, 0)


def _deg_body(dst_hbm, out_hbm, idx_v, idx_t, ones_v, ones_t, acc, zbuf):
    c = lax.axis_index("c")
    s = lax.axis_index("s")

    _zero_vmem(zbuf, RT)
    o = jnp.ones((16,), jnp.float32)

    def fill(i, _):
        ones_v[pl.ds(i * 16, 16)] = o
        return 0

    lax.fori_loop(0, CH // 16, fill, 0)

    def fill_t(i, _):
        ones_t[pl.ds(i * 16, 16)] = o
        return 0

    lax.fori_loop(0, TAILD // 16, fill_t, 0)

    pltpu.sync_copy(zbuf, acc.at[pl.ds(s * RT, RT)])
    plsc.subcore_barrier()

    ebase = (s * 2 + c) * ETD

    def chunk(j, _):
        pltpu.sync_copy(dst_hbm.at[pl.ds(ebase + j * CH, CH)], idx_v)
        pltpu.sync_copy(ones_v, acc.at[idx_v], add=True)
        return 0

    lax.fori_loop(0, NCHD, chunk, 0)
    pltpu.sync_copy(dst_hbm.at[pl.ds(ebase + NCHD * CH, TAILD)], idx_t)
    pltpu.sync_copy(ones_t, acc.at[idx_t], add=True)
    plsc.subcore_barrier()

    pltpu.sync_copy(acc.at[pl.ds(s * RT, RT)], out_hbm.at[c, pl.ds(s * RT, RT)])


@functools.cache
def _deg_partials():
    return pl.kernel(
    _deg_body,
    out_type=jax.ShapeDtypeStruct((2, NP), jnp.float32),
    mesh=_mesh(),
    scratch_types=[
        pltpu.VMEM((CH,), jnp.int32),
        pltpu.VMEM((TAILD,), jnp.int32),
        pltpu.VMEM((CH,), jnp.float32),
        pltpu.VMEM((TAILD,), jnp.float32),
        pltpu.VMEM_SHARED((NP,), jnp.float32),
        pltpu.VMEM((RT,), jnp.float32),
    ],
    compiler_params=_SC_PARAMS,
    )


_BLK = 640


def _mlp_body(x_ref, w1_ref, b1_ref, w2_ref, b2_ref, degp_ref,
              z01_ref, z02_ref, g1_ref, g2_ref, d16_ref):
    xb = x_ref[...]
    h = lax.dot_general(xb, w1_ref[...], (((1,), (1,)), ((), ())),
                        preferred_element_type=jnp.float32)
    h = jnp.maximum(h + b1_ref[...], 0.0)
    h2 = lax.dot_general(h, w2_ref[...], (((1,), (1,)), ((), ())),
                         preferred_element_type=jnp.float32)
    h2 = h2 + b2_ref[...]
    h2p = jnp.concatenate(
        [h2, jnp.zeros((_BLK, D2P - D2), jnp.float32)], axis=1)
    h2w = jnp.concatenate(
        [h2, jnp.zeros((_BLK, D1 - D2), jnp.float32)], axis=1)
    deg = 1.0 + degp_ref[0, :] + degp_ref[1, :]
    dis = lax.rsqrt(deg).reshape(_BLK, 1)
    z01_ref[...] = ALPHA * h
    z02_ref[...] = ALPHA * h2w
    g1_ref[...] = dis * h
    g2_ref[...] = dis * h2p
    d16_ref[...] = jnp.broadcast_to(dis, (_BLK, 16))


def _mlp(x, W1, b1, W2, b2, degp):
    n_blk = NP // _BLK
    return pl.pallas_call(
        _mlp_body,
        grid=(n_blk,),
        in_specs=[
            pl.BlockSpec((_BLK, 128), lambda i: (i, 0)),
            pl.BlockSpec((D1, 128), lambda i: (0, 0)),
            pl.BlockSpec((1, D1), lambda i: (0, 0)),
            pl.BlockSpec((D2, D1), lambda i: (0, 0)),
            pl.BlockSpec((1, D2), lambda i: (0, 0)),
            pl.BlockSpec((2, _BLK), lambda i: (0, i)),
        ],
        out_specs=[
            pl.BlockSpec((_BLK, D1), lambda i: (i, 0)),
            pl.BlockSpec((_BLK, D1), lambda i: (i, 0)),
            pl.BlockSpec((_BLK, D1), lambda i: (i, 0)),
            pl.BlockSpec((_BLK, D2P), lambda i: (i, 0)),
            pl.BlockSpec((_BLK, 16), lambda i: (i, 0)),
        ],
        out_shape=[
            jax.ShapeDtypeStruct((NP, D1), jnp.float32),
            jax.ShapeDtypeStruct((NP, D1), jnp.float32),
            jax.ShapeDtypeStruct((NP, D1), jnp.float32),
            jax.ShapeDtypeStruct((NP, D2P), jnp.float32),
            jax.ShapeDtypeStruct((NP, 16), jnp.float32),
        ],
    )(x, W1, b1, W2, b2, degp)


def _scatter_phase(src_hbm, dst_hbm, g_hbm, acc,
                   ss, dd, rr, semg, sems, semis, semid, semt,
                   sbuf_t, dbuf_t, rows_t, ebase):
    ntri = NCH // 3
    toff = ebase + NCH * CH

    pltpu.async_copy(src_hbm.at[pl.ds(toff, TAIL)], sbuf_t, semt)
    pltpu.async_copy(dst_hbm.at[pl.ds(toff, TAIL)], dbuf_t, semt)
    for k in range(3):
        off = ebase + k * CH
        pltpu.async_copy(src_hbm.at[pl.ds(off, CH)], ss[k], semis[k])
        pltpu.async_copy(dst_hbm.at[pl.ds(off, CH)], dd[k], semid[k])
    for k in range(3):
        off = ebase + k * CH
        pltpu.make_async_copy(src_hbm.at[pl.ds(off, CH)], ss[k],
                              semis[k]).wait()
        pltpu.async_copy(g_hbm.at[ss[k]], rr[k], semg[k])
    pltpu.make_async_copy(src_hbm.at[pl.ds(toff, TAIL)], sbuf_t, semt).wait()
    pltpu.make_async_copy(dst_hbm.at[pl.ds(toff, TAIL)], dbuf_t, semt).wait()
    pltpu.async_copy(g_hbm.at[sbuf_t], rows_t, semt)

    def tri(u, _):
        c0 = ebase + (3 * u) * CH

        for k in range(3):
            pltpu.make_async_copy(g_hbm.at[ss[k]], rr[k], semg[k]).wait()

            @pl.when(u < ntri - 1)
            def _():
                pltpu.async_copy(src_hbm.at[pl.ds(c0 + (3 + k) * CH, CH)],
                                 ss[k], semis[k])

            pltpu.make_async_copy(dst_hbm.at[pl.ds(c0, CH)], dd[k],
                                  semid[k]).wait()
            pltpu.async_copy(rr[k], acc.at[dd[k]], sems[k], add=True)

        for k in range(3):
            pltpu.make_async_copy(rr[k], acc.at[dd[k]], sems[k]).wait()

            @pl.when(u < ntri - 1)
            def _():
                off = c0 + (3 + k) * CH
                pltpu.async_copy(dst_hbm.at[pl.ds(off, CH)], dd[k], semid[k])
                pltpu.make_async_copy(src_hbm.at[pl.ds(off, CH)], ss[k],
                                      semis[k]).wait()
                pltpu.async_copy(g_hbm.at[ss[k]], rr[k], semg[k])

        return 0

    lax.fori_loop(0, ntri, tri, 0)
    pltpu.make_async_copy(g_hbm.at[sbuf_t], rows_t, semt).wait()
    pltpu.sync_copy(rows_t, acc.at[dbuf_t], add=True)


def _combine_rows(a, z, d16c, d, is_last):
    def row(r, _):
        vdis = d16c[r, :]
        for cc in range(d // 16):
            va = a[r, pl.ds(cc * 16, 16)]
            vz = z[r, pl.ds(cc * 16, 16)]
            vh = (1.0 - ALPHA) * (vdis * va) + vz
            vg = vdis * vh
            a[r, pl.ds(cc * 16, 16)] = jnp.where(is_last, vh, vg)
        return 0

    lax.fori_loop(0, CH, row, 0)


def _combine_phase(acc, z0_hbm, d16_hbm, g_hbm, ab, zb, d16c,
                   semz, semst, semse, rbase, is_last, d):
    nc = RT // CH
    pltpu.async_copy(z0_hbm.at[pl.ds(rbase, CH)], zb[0], semz[0])
    for j in range(nc):
        a = ab[j % 2]
        z = zb[j % 2]
        ro = rbase + j * CH
        if j >= 2:
            po = ro - 2 * CH
            pltpu.make_async_copy(a, g_hbm.at[pl.ds(po, CH)],
                                  semst[j % 2]).wait()
            pltpu.make_async_copy(a, acc.at[pl.ds(po, CH)],
                                  semse[j % 2]).wait()
        pltpu.sync_copy(acc.at[pl.ds(ro, CH)], a)
        if j < nc - 1:
            pltpu.async_copy(z0_hbm.at[pl.ds(ro + CH, CH)],
                             zb[(j + 1) % 2], semz[(j + 1) % 2])
        pltpu.sync_copy(d16_hbm.at[pl.ds(ro, CH)], d16c)
        pltpu.make_async_copy(z0_hbm.at[pl.ds(ro, CH)], z, semz[j % 2]).wait()
        _combine_rows(a, z, d16c, d, is_last)
        pltpu.async_copy(a, g_hbm.at[pl.ds(ro, CH)], semst[j % 2])
        pltpu.async_copy(a, acc.at[pl.ds(ro, CH)], semse[j % 2])
    for j in (nc - 2, nc - 1):
        ro = rbase + j * CH
        pltpu.make_async_copy(ab[j % 2], g_hbm.at[pl.ds(ro, CH)],
                              semst[j % 2]).wait()
        pltpu.make_async_copy(ab[j % 2], acc.at[pl.ds(ro, CH)],
                              semse[j % 2]).wait()


def _appnp_body(src_hbm, dst_hbm, z01_hbm, z02_hbm, g1_hbm, g2_hbm, d16_hbm,
                out1_hbm, out2_hbm,
                ss0, ss1, ss2, dd0, dd1, dd2,
                rA0, rA1, rA2, rB0, rB1, rB2, bufC,
                sbuf_t, dbuf_t, rows1_t, rows2_t, d16c,
                sg0, sg1, sg2, sc0, sc1, sc2,
                sis0, sis1, sis2, sid0, sid1, sid2, semt,
                sz0, sz1, sst0, sst1, sse0, sse1,
                acc1, acc2):
    ss = [ss0, ss1, ss2]
    dd = [dd0, dd1, dd2]
    rA = [rA0, rA1, rA2]
    rB = [rB0, rB1, rB2]
    semg = [sg0, sg1, sg2]
    sems = [sc0, sc1, sc2]
    semis = [sis0, sis1, sis2]
    semid = [sid0, sid1, sid2]
    semz = [sz0, sz1]
    semst = [sst0, sst1]
    semse = [sse0, sse1]
    zb = [rA2, bufC]
    c = lax.axis_index("c")
    s = lax.axis_index("s")
    rbase = s * RT
    ebase = s * ET
    on0 = c == 0
    on1 = c == 1

    @pl.when(on0)
    def _():
        def rchunk(j, _):
            ro = rbase + j * CH
            pltpu.sync_copy(g1_hbm.at[pl.ds(ro, CH)], rA0)
            pltpu.sync_copy(rA0, out1_hbm.at[pl.ds(ro, CH)])
            pltpu.sync_copy(rA0, acc1.at[pl.ds(ro, CH)])
            return 0

        lax.fori_loop(0, RT // CH, rchunk, 0)

    @pl.when(on1)
    def _():
        def rchunk(j, _):
            ro = rbase + j * CH
            pltpu.sync_copy(g2_hbm.at[pl.ds(ro, CH)], rB0)
            pltpu.sync_copy(rB0, out2_hbm.at[pl.ds(ro, CH)])
            pltpu.sync_copy(rB0, acc2.at[pl.ds(ro, CH)])
            return 0

        lax.fori_loop(0, RT // CH, rchunk, 0)

    def iteration(it, _):
        is_last = it == (K_IT - 1)
        plsc.subcore_barrier()

        @pl.when(on0)
        def _():
            _scatter_phase(src_hbm, dst_hbm, out1_hbm, acc1,
                           ss, dd, rA, semg, sems, semis, semid, semt,
                           sbuf_t, dbuf_t, rows1_t, ebase)

        @pl.when(on1)
        def _():
            _scatter_phase(src_hbm, dst_hbm, out2_hbm, acc2,
                           ss, dd, rB, semg, sems, semis, semid, semt,
                           sbuf_t, dbuf_t, rows2_t, ebase)

        plsc.subcore_barrier()

        @pl.when(on0)
        def _():
            _combine_phase(acc1, z01_hbm, d16_hbm, out1_hbm, [rA0, rA1],
                           zb, d16c, semz, semst, semse, rbase, is_last, D1)

        @pl.when(on1)
        def _():
            _combine_phase(acc2, z02_hbm, d16_hbm, out2_hbm, [rB0, rB1],
                           zb, d16c, semz, semst, semse, rbase, is_last, D2P)

        return 0

    lax.fori_loop(0, K_IT, iteration, 0)


@functools.cache
def _appnp():
    return pl.kernel(
    _appnp_body,
    out_type=(
        jax.ShapeDtypeStruct((NP, D1), jnp.float32),
        jax.ShapeDtypeStruct((NP, D2P), jnp.float32),
    ),
    mesh=_mesh(),
    scratch_types=(
        [pltpu.VMEM((CH,), jnp.int32)] * 6
        + [pltpu.VMEM((CH, D1), jnp.float32)] * 3
        + [pltpu.VMEM((CH, D2P), jnp.float32)] * 3
        + [pltpu.VMEM((CH, D1), jnp.float32)]
        + [
            pltpu.VMEM((TAIL,), jnp.int32),
            pltpu.VMEM((TAIL,), jnp.int32),
            pltpu.VMEM((TAIL, D1), jnp.float32),
            pltpu.VMEM((TAIL, D2P), jnp.float32),
            pltpu.VMEM((CH, 16), jnp.float32),
        ]
        + [pltpu.SemaphoreType.DMA] * 19
        + [
            pltpu.VMEM_SHARED((NP, D1), jnp.float32),
            pltpu.VMEM_SHARED((NP, D2P), jnp.float32),
        ]
    ),
    compiler_params=_SC_PARAMS,
    )


def kernel(x, edge_index, W1, b1, W2, b2):
    src = edge_index[0].astype(jnp.int32)
    dst = edge_index[1].astype(jnp.int32)
    xp = jnp.pad(x, ((0, NP - N), (0, 0)))
    degp = _deg_partials()(dst)
    z01, z02, g1, g2, d16 = _mlp(xp, W1, b1.reshape(1, D1), W2,
                                 b2.reshape(1, D2), degp)
    out1p, out2p = _appnp()(src, dst, z01, z02, g1, g2, d16)
    return (x, out1p[:N], out2p[:N, :D2])


# speedup vs baseline: 24.0387x; 1.0199x over previous
import functools

import jax
import jax.numpy as jnp
from jax import lax
from jax.experimental import pallas as pl
from jax.experimental.pallas import tpu as pltpu
from jax.experimental.pallas import tpu_sc as plsc

N = 10000
NP = 10240
E = 320000
K_IT = 10
ALPHA = 0.1
D1 = 64
D2 = 40
D2P = 48
RT = NP // 16
ET = E // 16
CH = 128
NCH = ET // CH
TAIL = ET - NCH * CH
ETD = E // 32
NCHD = ETD // CH
TAILD = ETD - NCHD * CH

_SC_PARAMS = pltpu.CompilerParams(use_tc_tiling_on_sc=False)


@functools.cache
def _mesh():
    return plsc.VectorSubcoreMesh(core_axis_name="c", subcore_axis_name="s")


def _zero_vmem(buf, n):
    z = jnp.zeros((16,), jnp.float32)

    def body(i, _):
        buf[pl.ds(i * 16, 16)] = z
        return 0

    lax.fori_loop(0, n // 16, body, 0)


def _deg_body(dst_hbm, out_hbm, dd0, dd1, dd2, idx_t, ones_v, ones_t,
              si0, si1, si2, sc0, sc1, sc2, acc, zbuf):
    c = lax.axis_index("c")
    s = lax.axis_index("s")
    dd = [dd0, dd1, dd2]
    semid = [si0, si1, si2]
    sems = [sc0, sc1, sc2]

    _zero_vmem(zbuf, RT)
    o = jnp.ones((16,), jnp.float32)

    def fill(i, _):
        ones_v[pl.ds(i * 16, 16)] = o
        return 0

    lax.fori_loop(0, CH // 16, fill, 0)

    def fill_t(i, _):
        ones_t[pl.ds(i * 16, 16)] = o
        return 0

    lax.fori_loop(0, TAILD // 16, fill_t, 0)

    pltpu.sync_copy(zbuf, acc.at[pl.ds(s * RT, RT)])
    plsc.subcore_barrier()

    ebase = (s * 2 + c) * ETD
    ntri = NCHD // 3

    for k in range(3):
        pltpu.async_copy(dst_hbm.at[pl.ds(ebase + k * CH, CH)], dd[k],
                         semid[k])

    def tri(u, _):
        c0 = ebase + (3 * u) * CH
        for k in range(3):
            pltpu.make_async_copy(dst_hbm.at[pl.ds(c0, CH)], dd[k],
                                  semid[k]).wait()
            pltpu.async_copy(ones_v, acc.at[dd[k]], sems[k], add=True)
        for k in range(3):
            pltpu.make_async_copy(ones_v, acc.at[dd[k]], sems[k]).wait()

            @pl.when(u < ntri - 1)
            def _():
                pltpu.async_copy(dst_hbm.at[pl.ds(c0 + (3 + k) * CH, CH)],
                                 dd[k], semid[k])

        return 0

    lax.fori_loop(0, ntri, tri, 0)
    pltpu.sync_copy(dst_hbm.at[pl.ds(ebase + NCHD * CH, TAILD)], idx_t)
    pltpu.sync_copy(ones_t, acc.at[idx_t], add=True)
    plsc.subcore_barrier()

    pltpu.sync_copy(acc.at[pl.ds(s * RT, RT)], out_hbm.at[c, pl.ds(s * RT, RT)])


@functools.cache
def _deg_partials():
    return pl.kernel(
    _deg_body,
    out_type=jax.ShapeDtypeStruct((2, NP), jnp.float32),
    mesh=_mesh(),
    scratch_types=[
        pltpu.VMEM((CH,), jnp.int32),
        pltpu.VMEM((CH,), jnp.int32),
        pltpu.VMEM((CH,), jnp.int32),
        pltpu.VMEM((TAILD,), jnp.int32),
        pltpu.VMEM((CH,), jnp.float32),
        pltpu.VMEM((TAILD,), jnp.float32),
        pltpu.SemaphoreType.DMA,
        pltpu.SemaphoreType.DMA,
        pltpu.SemaphoreType.DMA,
        pltpu.SemaphoreType.DMA,
        pltpu.SemaphoreType.DMA,
        pltpu.SemaphoreType.DMA,
        pltpu.VMEM_SHARED((NP,), jnp.float32),
        pltpu.VMEM((RT,), jnp.float32),
    ],
    compiler_params=_SC_PARAMS,
    )


_BLK = 640


def _mlp_body(x_ref, w1_ref, b1_ref, w2_ref, b2_ref, degp_ref,
              z01_ref, z02_ref, g1_ref, g2_ref, d16_ref):
    xb = x_ref[...]
    h = lax.dot_general(xb, w1_ref[...], (((1,), (1,)), ((), ())),
                        preferred_element_type=jnp.float32)
    h = jnp.maximum(h + b1_ref[...], 0.0)
    h2 = lax.dot_general(h, w2_ref[...], (((1,), (1,)), ((), ())),
                         preferred_element_type=jnp.float32)
    h2 = h2 + b2_ref[...]
    h2p = jnp.concatenate(
        [h2, jnp.zeros((_BLK, D2P - D2), jnp.float32)], axis=1)
    h2w = jnp.concatenate(
        [h2, jnp.zeros((_BLK, D1 - D2), jnp.float32)], axis=1)
    deg = 1.0 + degp_ref[0, :] + degp_ref[1, :]
    dis = lax.rsqrt(deg).reshape(_BLK, 1)
    z01_ref[...] = ALPHA * h
    z02_ref[...] = ALPHA * h2w
    g1_ref[...] = dis * h
    g2_ref[...] = dis * h2p
    d16_ref[...] = jnp.broadcast_to(dis, (_BLK, 16))


def _mlp(x, W1, b1, W2, b2, degp):
    n_blk = NP // _BLK
    return pl.pallas_call(
        _mlp_body,
        grid=(n_blk,),
        in_specs=[
            pl.BlockSpec((_BLK, 128), lambda i: (i, 0)),
            pl.BlockSpec((D1, 128), lambda i: (0, 0)),
            pl.BlockSpec((1, D1), lambda i: (0, 0)),
            pl.BlockSpec((D2, D1), lambda i: (0, 0)),
            pl.BlockSpec((1, D2), lambda i: (0, 0)),
            pl.BlockSpec((2, _BLK), lambda i: (0, i)),
        ],
        out_specs=[
            pl.BlockSpec((_BLK, D1), lambda i: (i, 0)),
            pl.BlockSpec((_BLK, D1), lambda i: (i, 0)),
            pl.BlockSpec((_BLK, D1), lambda i: (i, 0)),
            pl.BlockSpec((_BLK, D2P), lambda i: (i, 0)),
            pl.BlockSpec((_BLK, 16), lambda i: (i, 0)),
        ],
        out_shape=[
            jax.ShapeDtypeStruct((NP, D1), jnp.float32),
            jax.ShapeDtypeStruct((NP, D1), jnp.float32),
            jax.ShapeDtypeStruct((NP, D1), jnp.float32),
            jax.ShapeDtypeStruct((NP, D2P), jnp.float32),
            jax.ShapeDtypeStruct((NP, 16), jnp.float32),
        ],
    )(x, W1, b1, W2, b2, degp)


def _scatter_phase(src_hbm, dst_hbm, g_hbm, acc,
                   ss, dd, rr, semg, sems, semis, semid, semt,
                   sbuf_t, dbuf_t, rows_t, ebase):
    ntri = NCH // 3
    toff = ebase + NCH * CH

    pltpu.async_copy(src_hbm.at[pl.ds(toff, TAIL)], sbuf_t, semt)
    pltpu.async_copy(dst_hbm.at[pl.ds(toff, TAIL)], dbuf_t, semt)
    for k in range(3):
        off = ebase + k * CH
        pltpu.async_copy(src_hbm.at[pl.ds(off, CH)], ss[k], semis[k])
        pltpu.async_copy(dst_hbm.at[pl.ds(off, CH)], dd[k], semid[k])
    for k in range(3):
        off = ebase + k * CH
        pltpu.make_async_copy(src_hbm.at[pl.ds(off, CH)], ss[k],
                              semis[k]).wait()
        pltpu.async_copy(g_hbm.at[ss[k]], rr[k], semg[k])
    pltpu.make_async_copy(src_hbm.at[pl.ds(toff, TAIL)], sbuf_t, semt).wait()
    pltpu.make_async_copy(dst_hbm.at[pl.ds(toff, TAIL)], dbuf_t, semt).wait()
    pltpu.async_copy(g_hbm.at[sbuf_t], rows_t, semt)

    def tri(u, _):
        c0 = ebase + (3 * u) * CH

        for k in range(3):
            pltpu.make_async_copy(g_hbm.at[ss[k]], rr[k], semg[k]).wait()

            @pl.when(u < ntri - 1)
            def _():
                pltpu.async_copy(src_hbm.at[pl.ds(c0 + (3 + k) * CH, CH)],
                                 ss[k], semis[k])

            pltpu.make_async_copy(dst_hbm.at[pl.ds(c0, CH)], dd[k],
                                  semid[k]).wait()
            pltpu.async_copy(rr[k], acc.at[dd[k]], sems[k], add=True)

        for k in range(3):
            pltpu.make_async_copy(rr[k], acc.at[dd[k]], sems[k]).wait()

            @pl.when(u < ntri - 1)
            def _():
                off = c0 + (3 + k) * CH
                pltpu.async_copy(dst_hbm.at[pl.ds(off, CH)], dd[k], semid[k])
                pltpu.make_async_copy(src_hbm.at[pl.ds(off, CH)], ss[k],
                                      semis[k]).wait()
                pltpu.async_copy(g_hbm.at[ss[k]], rr[k], semg[k])

        return 0

    lax.fori_loop(0, ntri, tri, 0)
    pltpu.make_async_copy(g_hbm.at[sbuf_t], rows_t, semt).wait()
    pltpu.sync_copy(rows_t, acc.at[dbuf_t], add=True)


def _combine_rows(a, z, d16c, d, is_last):
    def row(r, _):
        vdis = d16c[r, :]
        for cc in range(d // 16):
            va = a[r, pl.ds(cc * 16, 16)]
            vz = z[r, pl.ds(cc * 16, 16)]
            vh = (1.0 - ALPHA) * (vdis * va) + vz
            vg = vdis * vh
            a[r, pl.ds(cc * 16, 16)] = jnp.where(is_last, vh, vg)
        return 0

    lax.fori_loop(0, CH, row, 0)


def _combine_phase(acc, z0_hbm, d16_hbm, g_hbm, ab, zb, d16c,
                   semz, semst, semse, rbase, is_last, d):
    nc = RT // CH
    pltpu.async_copy(z0_hbm.at[pl.ds(rbase, CH)], zb[0], semz[0])
    for j in range(nc):
        a = ab[j % 2]
        z = zb[j % 2]
        ro = rbase + j * CH
        if j >= 2:
            po = ro - 2 * CH
            pltpu.make_async_copy(a, g_hbm.at[pl.ds(po, CH)],
                                  semst[j % 2]).wait()
            pltpu.make_async_copy(a, acc.at[pl.ds(po, CH)],
                                  semse[j % 2]).wait()
        pltpu.sync_copy(acc.at[pl.ds(ro, CH)], a)
        if j < nc - 1:
            pltpu.async_copy(z0_hbm.at[pl.ds(ro + CH, CH)],
                             zb[(j + 1) % 2], semz[(j + 1) % 2])
        pltpu.sync_copy(d16_hbm.at[pl.ds(ro, CH)], d16c)
        pltpu.make_async_copy(z0_hbm.at[pl.ds(ro, CH)], z, semz[j % 2]).wait()
        _combine_rows(a, z, d16c, d, is_last)
        pltpu.async_copy(a, g_hbm.at[pl.ds(ro, CH)], semst[j % 2])
        pltpu.async_copy(a, acc.at[pl.ds(ro, CH)], semse[j % 2])
    for j in (nc - 2, nc - 1):
        ro = rbase + j * CH
        pltpu.make_async_copy(ab[j % 2], g_hbm.at[pl.ds(ro, CH)],
                              semst[j % 2]).wait()
        pltpu.make_async_copy(ab[j % 2], acc.at[pl.ds(ro, CH)],
                              semse[j % 2]).wait()


def _appnp_body(src_hbm, dst_hbm, z01_hbm, z02_hbm, g1_hbm, g2_hbm, d16_hbm,
                out1_hbm, out2_hbm,
                ss0, ss1, ss2, dd0, dd1, dd2,
                rA0, rA1, rA2, rB0, rB1, rB2, bufC,
                sbuf_t, dbuf_t, rows1_t, rows2_t, d16c,
                sg0, sg1, sg2, sc0, sc1, sc2,
                sis0, sis1, sis2, sid0, sid1, sid2, semt,
                sz0, sz1, sst0, sst1, sse0, sse1,
                acc1, acc2):
    ss = [ss0, ss1, ss2]
    dd = [dd0, dd1, dd2]
    rA = [rA0, rA1, rA2]
    rB = [rB0, rB1, rB2]
    semg = [sg0, sg1, sg2]
    sems = [sc0, sc1, sc2]
    semis = [sis0, sis1, sis2]
    semid = [sid0, sid1, sid2]
    semz = [sz0, sz1]
    semst = [sst0, sst1]
    semse = [sse0, sse1]
    zb = [rA2, bufC]
    c = lax.axis_index("c")
    s = lax.axis_index("s")
    rbase = s * RT
    ebase = s * ET
    on0 = c == 0
    on1 = c == 1

    @pl.when(on0)
    def _():
        def rchunk(j, _):
            ro = rbase + j * CH
            pltpu.sync_copy(g1_hbm.at[pl.ds(ro, CH)], rA0)
            pltpu.sync_copy(rA0, out1_hbm.at[pl.ds(ro, CH)])
            pltpu.sync_copy(rA0, acc1.at[pl.ds(ro, CH)])
            return 0

        lax.fori_loop(0, RT // CH, rchunk, 0)

    @pl.when(on1)
    def _():
        def rchunk(j, _):
            ro = rbase + j * CH
            pltpu.sync_copy(g2_hbm.at[pl.ds(ro, CH)], rB0)
            pltpu.sync_copy(rB0, out2_hbm.at[pl.ds(ro, CH)])
            pltpu.sync_copy(rB0, acc2.at[pl.ds(ro, CH)])
            return 0

        lax.fori_loop(0, RT // CH, rchunk, 0)

    def iteration(it, _):
        is_last = it == (K_IT - 1)
        plsc.subcore_barrier()

        @pl.when(on0)
        def _():
            _scatter_phase(src_hbm, dst_hbm, out1_hbm, acc1,
                           ss, dd, rA, semg, sems, semis, semid, semt,
                           sbuf_t, dbuf_t, rows1_t, ebase)

        @pl.when(on1)
        def _():
            _scatter_phase(src_hbm, dst_hbm, out2_hbm, acc2,
                           ss, dd, rB, semg, sems, semis, semid, semt,
                           sbuf_t, dbuf_t, rows2_t, ebase)

        plsc.subcore_barrier()

        @pl.when(on0)
        def _():
            _combine_phase(acc1, z01_hbm, d16_hbm, out1_hbm, [rA0, rA1],
                           zb, d16c, semz, semst, semse, rbase, is_last, D1)

        @pl.when(on1)
        def _():
            _combine_phase(acc2, z02_hbm, d16_hbm, out2_hbm, [rB0, rB1],
                           zb, d16c, semz, semst, semse, rbase, is_last, D2P)

        return 0

    lax.fori_loop(0, K_IT, iteration, 0)


@functools.cache
def _appnp():
    return pl.kernel(
    _appnp_body,
    out_type=(
        jax.ShapeDtypeStruct((NP, D1), jnp.float32),
        jax.ShapeDtypeStruct((NP, D2P), jnp.float32),
    ),
    mesh=_mesh(),
    scratch_types=(
        [pltpu.VMEM((CH,), jnp.int32)] * 6
        + [pltpu.VMEM((CH, D1), jnp.float32)] * 3
        + [pltpu.VMEM((CH, D2P), jnp.float32)] * 3
        + [pltpu.VMEM((CH, D1), jnp.float32)]
        + [
            pltpu.VMEM((TAIL,), jnp.int32),
            pltpu.VMEM((TAIL,), jnp.int32),
            pltpu.VMEM((TAIL, D1), jnp.float32),
            pltpu.VMEM((TAIL, D2P), jnp.float32),
            pltpu.VMEM((CH, 16), jnp.float32),
        ]
        + [pltpu.SemaphoreType.DMA] * 19
        + [
            pltpu.VMEM_SHARED((NP, D1), jnp.float32),
            pltpu.VMEM_SHARED((NP, D2P), jnp.float32),
        ]
    ),
    compiler_params=_SC_PARAMS,
    )


def kernel(x, edge_index, W1, b1, W2, b2):
    src = edge_index[0].astype(jnp.int32)
    dst = edge_index[1].astype(jnp.int32)
    xp = jnp.pad(x, ((0, NP - N), (0, 0)))
    degp = _deg_partials()(dst)
    z01, z02, g1, g2, d16 = _mlp(xp, W1, b1.reshape(1, D1), W2,
                                 b2.reshape(1, D2), degp)
    out1p, out2p = _appnp()(src, dst, z01, z02, g1, g2, d16)
    return (x, out1p[:N], out2p[:N, :D2])
